# baseline (speedup 1.0000x reference)
import functools

import jax
import jax.numpy as jnp
from jax.experimental import pallas as pl

B = 16384
N = 1000
ROWS = 512


def _onehot_body(x_ref, g_ref, o_ref):
    t = x_ref[...] + g_ref[...]
    m = jnp.max(t, axis=1, keepdims=True)
    e = jnp.exp(t - m)
    y = e / jnp.sum(e, axis=1, keepdims=True)
    m2 = jnp.max(y, axis=1, keepdims=True)
    col = jax.lax.broadcasted_iota(jnp.int32, t.shape, 1)
    first = jnp.min(jnp.where(y == m2, col, N), axis=1, keepdims=True)
    y_hard = (col == first).astype(jnp.float32)
    ret = y_hard - y + y
    out = jnp.where(ret > 0.5, ret, 0.0)
    row = jax.lax.broadcasted_iota(jnp.int32, t.shape, 0)
    is_row0 = (pl.program_id(0) == 0) & (row == 0) & (col == 1)
    o_ref[...] = jnp.where(is_row0, 1.0, out)


@jax.jit
def kernel(x, gumbels):
    return pl.pallas_call(
        _onehot_body,
        grid=(B // ROWS,),
        in_specs=[
            pl.BlockSpec((ROWS, N), lambda i: (i, 0)),
            pl.BlockSpec((ROWS, N), lambda i: (i, 0)),
        ],
        out_specs=pl.BlockSpec((ROWS, N), lambda i: (i, 0)),
        out_shape=jax.ShapeDtypeStruct((B, N), jnp.float32),
    )(x, gumbels)


# speedup vs baseline: 3.2636x; 3.2636x over previous
import jax
import jax.numpy as jnp
from jax.experimental import pallas as pl

B = 16384
N = 1000
COLS = 512


def _onehot_body(x_ref, g_ref, o_ref):
    t = x_ref[...] + g_ref[...]
    m = jnp.max(t, axis=0, keepdims=True)
    e = jnp.exp(t - m)
    y = e / jnp.sum(e, axis=0, keepdims=True)
    m2 = jnp.max(y, axis=0, keepdims=True)
    row = jax.lax.broadcasted_iota(jnp.int32, t.shape, 0)
    first = jnp.min(jnp.where(y == m2, row, N), axis=0, keepdims=True)
    y_hard = (row == first).astype(jnp.float32)
    ret = y_hard - y + y
    out = jnp.where(ret > 0.5, ret, 0.0)
    col = jax.lax.broadcasted_iota(jnp.int32, t.shape, 1)
    is_fix = (pl.program_id(0) == 0) & (row == 1) & (col == 0)
    o_ref[...] = jnp.where(is_fix, 1.0, out)


@jax.jit
def kernel(x, gumbels):
    out_t = pl.pallas_call(
        _onehot_body,
        grid=(B // COLS,),
        in_specs=[
            pl.BlockSpec((N, COLS), lambda i: (0, i)),
            pl.BlockSpec((N, COLS), lambda i: (0, i)),
        ],
        out_specs=pl.BlockSpec((N, COLS), lambda i: (0, i)),
        out_shape=jax.ShapeDtypeStruct((N, B), jnp.float32),
    )(x.T, gumbels.T)
    return out_t.T


# speedup vs baseline: 3.6743x; 1.1259x over previous
import jax
import jax.numpy as jnp
from jax.experimental import pallas as pl

B = 16384
N = 1000
COLS = 1024


def _onehot_body(x_ref, g_ref, o_ref):
    t = x_ref[...] + g_ref[...]
    m = jnp.max(t, axis=0, keepdims=True)
    e = jnp.exp(t - m)
    y = e / jnp.sum(e, axis=0, keepdims=True)
    m2 = jnp.max(y, axis=0, keepdims=True)
    row = jax.lax.broadcasted_iota(jnp.int32, t.shape, 0)
    first = jnp.min(jnp.where(y == m2, row, N), axis=0, keepdims=True)
    y_hard = (row == first).astype(jnp.float32)
    ret = y_hard - y + y
    out = jnp.where(ret > 0.5, ret, 0.0)
    col = jax.lax.broadcasted_iota(jnp.int32, t.shape, 1)
    is_fix = (pl.program_id(0) == 0) & (row == 1) & (col == 0)
    o_ref[...] = jnp.where(is_fix, 1.0, out)


@jax.jit
def kernel(x, gumbels):
    out_t = pl.pallas_call(
        _onehot_body,
        grid=(B // COLS,),
        in_specs=[
            pl.BlockSpec((N, COLS), lambda i: (0, i)),
            pl.BlockSpec((N, COLS), lambda i: (0, i)),
        ],
        out_specs=pl.BlockSpec((N, COLS), lambda i: (0, i)),
        out_shape=jax.ShapeDtypeStruct((N, B), jnp.float32),
    )(x.T, gumbels.T)
    return out_t.T
